# Initial kernel scaffold; baseline (speedup 1.0000x reference)
import jax
import jax.numpy as jnp
from jax.experimental import pallas as pl


def kernel(node_feat, pos, edge_index, edge_type, W_emb, etype_emb, W_filter, W_msg, W_upd, W_vgate, W_vmix):
    raise NotImplementedError("write your pallas kernel here")


# speedup vs baseline: 38.4164x; 38.4164x over previous
import functools

import numpy as _np

import jax
import jax.numpy as jnp
from jax import lax
from jax.experimental import pallas as pl
from jax.experimental.pallas import tpu as pltpu
from jax.experimental.pallas import tpu_sc as plsc

_N = 50000
_E = 800000
_FEAT = 27
_HS = 32
_HV = 8
_EC = 8
_NL = 6
_NT = 4
_CUTOFF = 10.0

_CB = 128
_NCHUNK = _E // _CB
_NTILES = 16
_CHUNK_PER = _NCHUNK // _NTILES
_CHUNK_REM = _NCHUNK % _NTILES
_ROWS_PER_TILE = _N // _NTILES

_BN = 2000
_BE = 2000


def _leaky(x):
    return jnp.where(x >= 0, x, 0.01 * x)


def _sc_mesh():
    return plsc.VectorSubcoreMesh(core_axis_name="c", subcore_axis_name="s")


def _gather_pos(pos4, src, dst):
    @functools.partial(
        pl.kernel,
        mesh=_sc_mesh(),
        compiler_params=pltpu.CompilerParams(use_tc_tiling_on_sc=False),
        out_type=[
            jax.ShapeDtypeStruct((_E, 4), jnp.float32),
            jax.ShapeDtypeStruct((_E, 4), jnp.float32),
        ],
        scratch_types=[
            pltpu.VMEM((_CB,), jnp.int32),
            pltpu.VMEM((_CB, 4), jnp.float32),
        ],
    )
    def k(pos4_hbm, src_hbm, dst_hbm, psrc_hbm, pdst_hbm, idx_v, rows_v):
        c = lax.axis_index("c")
        s = lax.axis_index("s")
        ntr = jnp.where(s < _CHUNK_REM, _CHUNK_PER + 1, _CHUNK_PER)

        def body(i, carry):
            base = (s + i * _NTILES) * _CB

            @pl.when(c == 0)
            def _():
                pltpu.sync_copy(src_hbm.at[pl.ds(base, _CB)], idx_v)
                pltpu.sync_copy(pos4_hbm.at[idx_v], rows_v)
                pltpu.sync_copy(rows_v, psrc_hbm.at[pl.ds(base, _CB)])

            @pl.when(c == 1)
            def _():
                pltpu.sync_copy(dst_hbm.at[pl.ds(base, _CB)], idx_v)
                pltpu.sync_copy(pos4_hbm.at[idx_v], rows_v)
                pltpu.sync_copy(rows_v, pdst_hbm.at[pl.ds(base, _CB)])

            return carry

        lax.fori_loop(0, ntr, body, 0)

    return k(pos4, src, dst)


def _edge_aggregate(gs, gv, fc, u, src, dst, zeros32):
    @functools.partial(
        pl.kernel,
        mesh=_sc_mesh(),
        compiler_params=pltpu.CompilerParams(use_tc_tiling_on_sc=False),
        out_type=[
            jax.ShapeDtypeStruct((_N, 32), jnp.float32),
            jax.ShapeDtypeStruct((_N, 32), jnp.float32),
        ],
        scratch_types=[
            pltpu.VMEM_SHARED((_N, 32), jnp.float32),
            pltpu.VMEM((_CB,), jnp.int32),
            pltpu.VMEM((_CB,), jnp.int32),
            pltpu.VMEM((_CB, 32), jnp.float32),
            pltpu.VMEM((_CB, 32), jnp.float32),
            pltpu.VMEM((_CB, 48), jnp.float32),
            pltpu.VMEM((_CB, 48), jnp.float32),
            pltpu.VMEM((_CB, 32), jnp.float32),
        ],
    )
    def k(gs_hbm, gv_hbm, fc_hbm, u_hbm, src_hbm, dst_hbm, zero_hbm,
          accs_hbm, accv_hbm,
          acc_sh, src_v, dst_v, gs_rows, fc_rows, gv_rows, u_rows, msg):
        c = lax.axis_index("c")
        s = lax.axis_index("s")
        r0 = s * _ROWS_PER_TILE
        pltpu.sync_copy(zero_hbm.at[pl.ds(r0, _ROWS_PER_TILE)],
                        acc_sh.at[pl.ds(r0, _ROWS_PER_TILE)])
        plsc.subcore_barrier()

        ntr = jnp.where(s < _CHUNK_REM, _CHUNK_PER + 1, _CHUNK_PER)
        mask8 = lax.iota(jnp.int32, 16) < 8

        def body(i, carry):
            base = (s + i * _NTILES) * _CB
            pltpu.sync_copy(src_hbm.at[pl.ds(base, _CB)], src_v)
            pltpu.sync_copy(dst_hbm.at[pl.ds(base, _CB)], dst_v)

            @pl.when(c == 0)
            def _():
                pltpu.sync_copy(gs_hbm.at[src_v], gs_rows)
                pltpu.sync_copy(fc_hbm.at[pl.ds(base, _CB)], fc_rows)

                def ebody(e, cc):
                    msg[e, pl.ds(0, 16)] = (gs_rows[e, pl.ds(0, 16)] *
                                            fc_rows[e, pl.ds(0, 16)])
                    msg[e, pl.ds(16, 16)] = (gs_rows[e, pl.ds(16, 16)] *
                                             fc_rows[e, pl.ds(16, 16)])
                    return cc

                lax.fori_loop(0, _CB, ebody, 0)

            @pl.when(c == 1)
            def _():
                pltpu.sync_copy(gv_hbm.at[src_v], gv_rows)
                pltpu.sync_copy(u_hbm.at[pl.ds(base, _CB)], u_rows)

                def ebody(e, cc):
                    a0 = gv_rows[e, pl.ds(0, 16)]
                    a1 = gv_rows[e, pl.ds(16, 16)]
                    a2 = gv_rows[e, pl.ds(32, 16)]
                    u0 = u_rows[e, pl.ds(0, 16)]
                    u1 = u_rows[e, pl.ds(16, 16)]
                    u2 = u_rows[e, pl.ds(32, 16)]
                    o0 = a0 * u2 + u0 * a2
                    o1 = a1 * u2 + u1 * a2
                    o1 = jnp.where(mask8, o1, 0.0)
                    msg[e, pl.ds(0, 16)] = o0
                    msg[e, pl.ds(16, 16)] = o1
                    return cc

                lax.fori_loop(0, _CB, ebody, 0)

            pltpu.sync_copy(msg, acc_sh.at[dst_v], add=True)
            return carry

        lax.fori_loop(0, ntr, body, 0)
        plsc.subcore_barrier()

        @pl.when(c == 0)
        def _():
            pltpu.sync_copy(acc_sh.at[pl.ds(r0, _ROWS_PER_TILE)],
                            accs_hbm.at[pl.ds(r0, _ROWS_PER_TILE)])

        @pl.when(c == 1)
        def _():
            pltpu.sync_copy(acc_sh.at[pl.ds(r0, _ROWS_PER_TILE)],
                            accv_hbm.at[pl.ds(r0, _ROWS_PER_TILE)])

    return k(gs, gv, fc, u, src, dst, zeros32)


def _tc_init(node_feat, W_emb, W_msg0, W_vgate0):
    def body(nf_ref, we_ref, wm_ref, wg_ref, h_ref, gs_ref, gv_ref):
        h = jnp.dot(nf_ref[...], we_ref[...],
                    preferred_element_type=jnp.float32)
        h_ref[...] = h
        gs_ref[...] = jnp.dot(h, wm_ref[...],
                              preferred_element_type=jnp.float32)
        g = jax.nn.sigmoid(jnp.dot(h, wg_ref[...],
                                   preferred_element_type=jnp.float32))
        z24 = jnp.zeros((_BN, 24), jnp.float32)
        gv_ref[...] = jnp.concatenate([z24, g, g, g], axis=1)

    return pl.pallas_call(
        body,
        grid=(_N // _BN,),
        in_specs=[
            pl.BlockSpec((_BN, _FEAT), lambda i: (i, 0)),
            pl.BlockSpec((_FEAT, _HS), lambda i: (0, 0)),
            pl.BlockSpec((_HS, _HS), lambda i: (0, 0)),
            pl.BlockSpec((_HS, _HV), lambda i: (0, 0)),
        ],
        out_specs=[
            pl.BlockSpec((_BN, _HS), lambda i: (i, 0)),
            pl.BlockSpec((_BN, _HS), lambda i: (i, 0)),
            pl.BlockSpec((_BN, 48), lambda i: (i, 0)),
        ],
        out_shape=[
            jax.ShapeDtypeStruct((_N, _HS), jnp.float32),
            jax.ShapeDtypeStruct((_N, _HS), jnp.float32),
            jax.ShapeDtypeStruct((_N, 48), jnp.float32),
        ],
    )(node_feat, W_emb, W_msg0, W_vgate0)


def _tc_edgeprep(psrc, pdst, etype2, etype_emb, W_filter):
    def body(ps_ref, pd_ref, et_ref, ee_ref, wf_ref, fc_ref, u_ref):
        rel = pd_ref[...] - ps_ref[...]
        r2 = jnp.sum(rel * rel, axis=1, keepdims=True)
        d = jnp.sqrt(r2 + 1e-12)
        unit = rel / d
        mus = _np.linspace(0.0, _CUTOFF, _EC)
        gamma = 1.0 / (2.0 * (_CUTOFF / _EC) ** 2)
        dm = jnp.concatenate([d - float(m) for m in mus], axis=1)
        rbf = jnp.exp(-gamma * dm * dm)
        cc = 0.5 * (jnp.cos(jnp.pi * jnp.clip(d, 0.0, _CUTOFF) / _CUTOFF)
                    + 1.0)
        wf = wf_ref[...]
        w2 = jnp.dot(ee_ref[...], wf[_EC:, :],
                     preferred_element_type=jnp.float32)
        onehot = (et_ref[...] == jnp.arange(_NT, dtype=jnp.int32)[None, :]
                  ).astype(jnp.float32)
        pre = (jnp.dot(rbf, wf[:_EC, :], preferred_element_type=jnp.float32)
               + jnp.dot(onehot, w2, preferred_element_type=jnp.float32))
        fc_ref[...] = _leaky(pre) * cc
        uc = unit * cc
        e8 = jnp.ones((1, 8), jnp.float32)
        u_ref[...] = jnp.concatenate(
            [uc[:, 0:1] * e8, uc[:, 1:2] * e8, uc[:, 2:3] * e8,
             jnp.zeros((_BE, 8), jnp.float32), cc * e8, cc * e8], axis=1)

    return pl.pallas_call(
        body,
        grid=(_E // _BE,),
        in_specs=[
            pl.BlockSpec((_BE, 4), lambda i: (i, 0)),
            pl.BlockSpec((_BE, 4), lambda i: (i, 0)),
            pl.BlockSpec((_BE, 1), lambda i: (i, 0)),
            pl.BlockSpec((_NT, _EC), lambda i: (0, 0)),
            pl.BlockSpec((2 * _EC, _HS), lambda i: (0, 0)),
        ],
        out_specs=[
            pl.BlockSpec((_BE, _HS), lambda i: (i, 0)),
            pl.BlockSpec((_BE, 48), lambda i: (i, 0)),
        ],
        out_shape=[
            jax.ShapeDtypeStruct((_E, _HS), jnp.float32),
            jax.ShapeDtypeStruct((_E, 48), jnp.float32),
        ],
    )(psrc, pdst, etype2, etype_emb, W_filter)


def _tc_update_mid(h, hv, aggs, aggv, W_upd_l, Wvmix3_l, W_msg_n, W_vgate_n):
    def body(h_ref, hv_ref, as_ref, av_ref, wu_ref, wm3_ref, wm_ref, wg_ref,
             h2_ref, hv2_ref, gs_ref, gv_ref):
        h2 = h_ref[...] + _leaky(jnp.dot(as_ref[...], wu_ref[...],
                                         preferred_element_type=jnp.float32))
        t = hv_ref[...] + av_ref[...][:, 0:24]
        hv2 = jnp.dot(t, wm3_ref[...], preferred_element_type=jnp.float32)
        h2_ref[...] = h2
        hv2_ref[...] = hv2
        gs_ref[...] = jnp.dot(h2, wm_ref[...],
                              preferred_element_type=jnp.float32)
        g = jax.nn.sigmoid(jnp.dot(h2, wg_ref[...],
                                   preferred_element_type=jnp.float32))
        g3 = jnp.concatenate([g, g, g], axis=1)
        gv_ref[...] = jnp.concatenate([hv2 * g3, g3], axis=1)

    return pl.pallas_call(
        body,
        grid=(_N // _BN,),
        in_specs=[
            pl.BlockSpec((_BN, _HS), lambda i: (i, 0)),
            pl.BlockSpec((_BN, 24), lambda i: (i, 0)),
            pl.BlockSpec((_BN, _HS), lambda i: (i, 0)),
            pl.BlockSpec((_BN, _HS), lambda i: (i, 0)),
            pl.BlockSpec((_HS, _HS), lambda i: (0, 0)),
            pl.BlockSpec((24, 24), lambda i: (0, 0)),
            pl.BlockSpec((_HS, _HS), lambda i: (0, 0)),
            pl.BlockSpec((_HS, _HV), lambda i: (0, 0)),
        ],
        out_specs=[
            pl.BlockSpec((_BN, _HS), lambda i: (i, 0)),
            pl.BlockSpec((_BN, 24), lambda i: (i, 0)),
            pl.BlockSpec((_BN, _HS), lambda i: (i, 0)),
            pl.BlockSpec((_BN, 48), lambda i: (i, 0)),
        ],
        out_shape=[
            jax.ShapeDtypeStruct((_N, _HS), jnp.float32),
            jax.ShapeDtypeStruct((_N, 24), jnp.float32),
            jax.ShapeDtypeStruct((_N, _HS), jnp.float32),
            jax.ShapeDtypeStruct((_N, 48), jnp.float32),
        ],
    )(h, hv, aggs, aggv, W_upd_l, Wvmix3_l, W_msg_n, W_vgate_n)


def _tc_update_last(h, hv, aggs, aggv, W_upd_l, Wvmix3_l):
    def body(h_ref, hv_ref, as_ref, av_ref, wu_ref, wm3_ref, out_ref):
        h2 = h_ref[...] + _leaky(jnp.dot(as_ref[...], wu_ref[...],
                                         preferred_element_type=jnp.float32))
        t = hv_ref[...] + av_ref[...][:, 0:24]
        hv2 = jnp.dot(t, wm3_ref[...], preferred_element_type=jnp.float32)
        cols = [hv2[:, i * _HV + v:i * _HV + v + 1]
                for v in range(_HV) for i in range(3)]
        out_ref[...] = jnp.concatenate([h2] + cols, axis=1)

    return pl.pallas_call(
        body,
        grid=(_N // _BN,),
        in_specs=[
            pl.BlockSpec((_BN, _HS), lambda i: (i, 0)),
            pl.BlockSpec((_BN, 24), lambda i: (i, 0)),
            pl.BlockSpec((_BN, _HS), lambda i: (i, 0)),
            pl.BlockSpec((_BN, _HS), lambda i: (i, 0)),
            pl.BlockSpec((_HS, _HS), lambda i: (0, 0)),
            pl.BlockSpec((24, 24), lambda i: (0, 0)),
        ],
        out_specs=pl.BlockSpec((_BN, 56), lambda i: (i, 0)),
        out_shape=jax.ShapeDtypeStruct((_N, 56), jnp.float32),
    )(h, hv, aggs, aggv, W_upd_l, Wvmix3_l)


def kernel(node_feat, pos, edge_index, edge_type, W_emb, etype_emb, W_filter,
           W_msg, W_upd, W_vgate, W_vmix):
    src = edge_index[0].astype(jnp.int32)
    dst = edge_index[1].astype(jnp.int32)
    etype2 = edge_type.astype(jnp.int32).reshape(_E, 1)
    pos4 = jnp.pad(pos, ((0, 0), (0, 1)))
    zeros32 = jnp.zeros((_N, 32), jnp.float32)
    eye3 = jnp.eye(3, dtype=jnp.float32)
    Wvmix3 = jax.vmap(lambda w: jnp.kron(eye3, w))(W_vmix)

    psrc, pdst = _gather_pos(pos4, src, dst)
    fc, u = _tc_edgeprep(psrc, pdst, etype2, etype_emb, W_filter)
    h, gs, gv = _tc_init(node_feat, W_emb, W_msg[0], W_vgate[0])
    hv = jnp.zeros((_N, 24), jnp.float32)

    for l in range(_NL):
        aggs, aggv = _edge_aggregate(gs, gv, fc, u, src, dst, zeros32)
        if l + 1 < _NL:
            h, hv, gs, gv = _tc_update_mid(
                h, hv, aggs, aggv, W_upd[l], Wvmix3[l],
                W_msg[l + 1], W_vgate[l + 1])
        else:
            out = _tc_update_last(h, hv, aggs, aggv, W_upd[l], Wvmix3[l])
    return out


# speedup vs baseline: 63.8660x; 1.6625x over previous
import functools

import numpy as _np

import jax
import jax.numpy as jnp
from jax import lax
from jax.experimental import pallas as pl
from jax.experimental.pallas import tpu as pltpu
from jax.experimental.pallas import tpu_sc as plsc

_N = 50000
_E = 800000
_FEAT = 27
_HS = 32
_HV = 8
_EC = 8
_NL = 6
_NT = 4
_CUTOFF = 10.0

_CB = 125
_NCHUNK = _E // _CB
_NTILES = 16
_CHUNK_PER = _NCHUNK // _NTILES
_ROWS_PER_TILE = _N // _NTILES

_BN = 2000
_BE = 2000


def _leaky(x):
    return jnp.where(x >= 0, x, 0.01 * x)


def _sc_mesh():
    return plsc.VectorSubcoreMesh(core_axis_name="c", subcore_axis_name="s")


def _gather_pos(pos4, src, dst):
    src2 = src.reshape(_NCHUNK, _CB)
    dst2 = dst.reshape(_NCHUNK, _CB)

    @functools.partial(
        pl.kernel,
        mesh=_sc_mesh(),
        compiler_params=pltpu.CompilerParams(use_tc_tiling_on_sc=False),
        out_type=[
            jax.ShapeDtypeStruct((_E, 4), jnp.float32),
            jax.ShapeDtypeStruct((_E, 4), jnp.float32),
        ],
        scratch_types=[
            pltpu.VMEM((_CB,), jnp.int32),
            pltpu.VMEM((_CB,), jnp.int32),
            pltpu.VMEM((_CB, 4), jnp.float32),
            pltpu.VMEM((_CB, 4), jnp.float32),
            pltpu.SemaphoreType.DMA,
            pltpu.SemaphoreType.DMA,
            pltpu.SemaphoreType.DMA,
            pltpu.SemaphoreType.DMA,
            pltpu.SemaphoreType.DMA,
            pltpu.SemaphoreType.DMA,
        ],
    )
    def k(pos4_hbm, src_hbm, dst_hbm, psrc_hbm, pdst_hbm,
          idx0, idx1, rows0, rows1, si0, si1, sg0, sg1, sw0, sw1):
        c = lax.axis_index("c")
        s = lax.axis_index("s")
        c0 = s * _CHUNK_PER
        idxs = (idx0, idx1)
        rows = (rows0, rows1)
        si = (si0, si1)
        sg = (sg0, sg1)
        sw = (sw0, sw1)

        def run(ehbm, out_hbm):
            def issue_idx(j, b):
                pltpu.async_copy(ehbm.at[j], idxs[b], si[b])

            def wait_idx(j, b):
                pltpu.make_async_copy(ehbm.at[j], idxs[b], si[b]).wait()

            def issue_g(b):
                pltpu.async_copy(pos4_hbm.at[idxs[b]], rows[b], sg[b])

            def wait_g(b):
                pltpu.make_async_copy(pos4_hbm.at[idxs[b]], rows[b],
                                      sg[b]).wait()

            def issue_w(j, b):
                pltpu.async_copy(rows[b],
                                 out_hbm.at[pl.ds(j * _CB, _CB)], sw[b])

            def wait_w(j, b):
                pltpu.make_async_copy(rows[b],
                                      out_hbm.at[pl.ds(j * _CB, _CB)],
                                      sw[b]).wait()

            issue_idx(c0, 0)
            issue_idx(c0 + 1, 1)
            wait_idx(c0, 0)
            issue_g(0)

            def outer(g, carry):
                for b in (0, 1):
                    i = g * 2 + b
                    nb = 1 - b

                    def nxt():
                        wait_idx(c0 + i + 1, nb)
                        @pl.when(i >= 1)
                        def _():
                            wait_w(c0 + i - 1, nb)
                        issue_g(nb)

                    if b == 0:
                        nxt()
                    else:
                        @pl.when(g < _CHUNK_PER // 2 - 1)
                        def _():
                            nxt()
                    wait_g(b)

                    @pl.when(g < _CHUNK_PER // 2 - 1)
                    def _():
                        issue_idx(c0 + i + 2, b)

                    issue_w(c0 + i, b)
                return carry

            lax.fori_loop(0, _CHUNK_PER // 2, outer, 0)
            wait_w(c0 + _CHUNK_PER - 2, 0)
            wait_w(c0 + _CHUNK_PER - 1, 1)

        @pl.when(c == 0)
        def _():
            run(src_hbm, psrc_hbm)

        @pl.when(c == 1)
        def _():
            run(dst_hbm, pdst_hbm)

    return k(pos4, src2, dst2)


def _edge_aggregate(gs, gv32, gv16, fc, u32, u16, src, dst, zeros32):
    src2 = src.reshape(_NCHUNK, _CB)
    dst2 = dst.reshape(_NCHUNK, _CB)

    @functools.partial(
        pl.kernel,
        mesh=_sc_mesh(),
        compiler_params=pltpu.CompilerParams(use_tc_tiling_on_sc=False),
        out_type=[
            jax.ShapeDtypeStruct((_N, 32), jnp.float32),
            jax.ShapeDtypeStruct((_N, 32), jnp.float32),
        ],
        scratch_types=[
            pltpu.VMEM_SHARED((_N, 32), jnp.float32),
            pltpu.VMEM((_CB,), jnp.int32),
            pltpu.VMEM((_CB,), jnp.int32),
            pltpu.VMEM((_CB,), jnp.int32),
            pltpu.VMEM((_CB,), jnp.int32),
            pltpu.VMEM((_CB, 32), jnp.float32),
            pltpu.VMEM((_CB, 32), jnp.float32),
            pltpu.VMEM((_CB, 16), jnp.float32),
            pltpu.VMEM((_CB, 16), jnp.float32),
            pltpu.VMEM((_CB, 32), jnp.float32),
            pltpu.VMEM((_CB, 32), jnp.float32),
            pltpu.VMEM((_CB, 16), jnp.float32),
            pltpu.VMEM((_CB, 16), jnp.float32),
            pltpu.VMEM((_CB, 32), jnp.float32),
            pltpu.SemaphoreType.DMA,
            pltpu.SemaphoreType.DMA,
            pltpu.SemaphoreType.DMA,
            pltpu.SemaphoreType.DMA,
            pltpu.SemaphoreType.DMA,
            pltpu.SemaphoreType.DMA,
        ],
    )
    def k(gs_hbm, gv32_hbm, gv16_hbm, fc_hbm, u32_hbm, u16_hbm,
          src_hbm, dst_hbm, zero_hbm,
          accs_hbm, accv_hbm,
          acc_sh, srcb0, srcb1, dstb0, dstb1,
          rows0, rows1, rx0, rx1, lin0, lin1, lx0, lx1, msg,
          si0, si1, sg0, sg1, sl0, sl1):
        c = lax.axis_index("c")
        s = lax.axis_index("s")
        r0 = s * _ROWS_PER_TILE
        pltpu.sync_copy(zero_hbm.at[pl.ds(r0, _ROWS_PER_TILE)],
                        acc_sh.at[pl.ds(r0, _ROWS_PER_TILE)])
        plsc.subcore_barrier()

        c0 = s * _CHUNK_PER
        srcb = (srcb0, srcb1)
        dstb = (dstb0, dstb1)
        rows = (rows0, rows1)
        rx = (rx0, rx1)
        lin = (lin0, lin1)
        lx = (lx0, lx1)
        si = (si0, si1)
        sg = (sg0, sg1)
        sl = (sl0, sl1)
        mask8 = lax.iota(jnp.int32, 16) < 8

        def run(gat_hbm, gx_hbm, lin_hbm, lnx_hbm, compute_chunk):
            def issue_idx_src(j, b):
                pltpu.async_copy(src_hbm.at[j], srcb[b], si[b])

            def issue_idx_dst(j, b):
                pltpu.async_copy(dst_hbm.at[j], dstb[b], si[b])

            def wait_idx(j, b):
                pltpu.make_async_copy(src_hbm.at[j], srcb[b], si[b]).wait()
                pltpu.make_async_copy(dst_hbm.at[j], dstb[b], si[b]).wait()

            def issue_g(j, b):
                pltpu.async_copy(gat_hbm.at[srcb[b]], rows[b], sg[b])
                pltpu.async_copy(lin_hbm.at[pl.ds(j * _CB, _CB)],
                                 lin[b], sl[b])
                if gx_hbm is not None:
                    pltpu.async_copy(gx_hbm.at[srcb[b]], rx[b], sg[b])
                    pltpu.async_copy(lnx_hbm.at[pl.ds(j * _CB, _CB)],
                                     lx[b], sl[b])

            def wait_g(j, b):
                pltpu.make_async_copy(gat_hbm.at[srcb[b]], rows[b],
                                      sg[b]).wait()
                pltpu.make_async_copy(lin_hbm.at[pl.ds(j * _CB, _CB)],
                                      lin[b], sl[b]).wait()
                if gx_hbm is not None:
                    pltpu.make_async_copy(gx_hbm.at[srcb[b]], rx[b],
                                          sg[b]).wait()
                    pltpu.make_async_copy(lnx_hbm.at[pl.ds(j * _CB, _CB)],
                                          lx[b], sl[b]).wait()

            issue_idx_src(c0, 0)
            issue_idx_dst(c0, 0)
            issue_idx_src(c0 + 1, 1)
            issue_idx_dst(c0 + 1, 1)
            wait_idx(c0, 0)
            issue_g(c0, 0)

            def outer(g, carry):
                for b in (0, 1):
                    i = g * 2 + b
                    nb = 1 - b

                    def nxt():
                        wait_idx(c0 + i + 1, nb)
                        issue_g(c0 + i + 1, nb)

                    if b == 0:
                        nxt()
                    else:
                        @pl.when(g < _CHUNK_PER // 2 - 1)
                        def _():
                            nxt()
                    wait_g(c0 + i, b)

                    @pl.when(g < _CHUNK_PER // 2 - 1)
                    def _():
                        issue_idx_src(c0 + i + 2, b)

                    compute_chunk(rows[b], rx[b], lin[b], lx[b])
                    pltpu.sync_copy(msg, acc_sh.at[dstb[b]], add=True)

                    @pl.when(g < _CHUNK_PER // 2 - 1)
                    def _():
                        issue_idx_dst(c0 + i + 2, b)
                return carry

            lax.fori_loop(0, _CHUNK_PER // 2, outer, 0)

        def compute_sca(rows_b, rx_b, lin_b, lx_b):
            def body(e5, cc):
                e0 = e5 * 5
                for t in range(5):
                    e = e0 + t
                    msg[e, pl.ds(0, 16)] = (rows_b[e, pl.ds(0, 16)] *
                                            lin_b[e, pl.ds(0, 16)])
                    msg[e, pl.ds(16, 16)] = (rows_b[e, pl.ds(16, 16)] *
                                             lin_b[e, pl.ds(16, 16)])
                return cc

            lax.fori_loop(0, _CB // 5, body, 0)

        def compute_vec(rows_b, rx_b, lin_b, lx_b):
            def body(e5, cc):
                e0 = e5 * 5
                for t in range(5):
                    e = e0 + t
                    a2 = rx_b[e, pl.ds(0, 16)]
                    u2 = lx_b[e, pl.ds(0, 16)]
                    o0 = (rows_b[e, pl.ds(0, 16)] * u2 +
                          lin_b[e, pl.ds(0, 16)] * a2)
                    o1 = (rows_b[e, pl.ds(16, 16)] * u2 +
                          lin_b[e, pl.ds(16, 16)] * a2)
                    msg[e, pl.ds(0, 16)] = o0
                    msg[e, pl.ds(16, 16)] = jnp.where(mask8, o1, 0.0)
                return cc

            lax.fori_loop(0, _CB // 5, body, 0)

        @pl.when(c == 0)
        def _():
            run(gs_hbm, None, fc_hbm, None, compute_sca)

        @pl.when(c == 1)
        def _():
            run(gv32_hbm, gv16_hbm, u32_hbm, u16_hbm, compute_vec)

        plsc.subcore_barrier()

        @pl.when(c == 0)
        def _():
            pltpu.sync_copy(acc_sh.at[pl.ds(r0, _ROWS_PER_TILE)],
                            accs_hbm.at[pl.ds(r0, _ROWS_PER_TILE)])

        @pl.when(c == 1)
        def _():
            pltpu.sync_copy(acc_sh.at[pl.ds(r0, _ROWS_PER_TILE)],
                            accv_hbm.at[pl.ds(r0, _ROWS_PER_TILE)])

    return k(gs, gv32, gv16, fc, u32, u16, src2, dst2, zeros32)


def _tc_init(node_feat, W_emb, W_msg0, W_vgate0):
    def body(nf_ref, we_ref, wm_ref, wg_ref, h_ref, gs_ref, gv32_ref,
             gv16_ref):
        h = jnp.dot(nf_ref[...], we_ref[...],
                    preferred_element_type=jnp.float32)
        h_ref[...] = h
        gs_ref[...] = jnp.dot(h, wm_ref[...],
                              preferred_element_type=jnp.float32)
        g = jax.nn.sigmoid(jnp.dot(h, wg_ref[...],
                                   preferred_element_type=jnp.float32))
        z24 = jnp.zeros((_BN, 24), jnp.float32)
        gv32_ref[...] = jnp.concatenate([z24, g], axis=1)
        gv16_ref[...] = jnp.concatenate([g, g], axis=1)

    return pl.pallas_call(
        body,
        grid=(_N // _BN,),
        in_specs=[
            pl.BlockSpec((_BN, _FEAT), lambda i: (i, 0)),
            pl.BlockSpec((_FEAT, _HS), lambda i: (0, 0)),
            pl.BlockSpec((_HS, _HS), lambda i: (0, 0)),
            pl.BlockSpec((_HS, _HV), lambda i: (0, 0)),
        ],
        out_specs=[
            pl.BlockSpec((_BN, _HS), lambda i: (i, 0)),
            pl.BlockSpec((_BN, _HS), lambda i: (i, 0)),
            pl.BlockSpec((_BN, 32), lambda i: (i, 0)),
            pl.BlockSpec((_BN, 16), lambda i: (i, 0)),
        ],
        out_shape=[
            jax.ShapeDtypeStruct((_N, _HS), jnp.float32),
            jax.ShapeDtypeStruct((_N, _HS), jnp.float32),
            jax.ShapeDtypeStruct((_N, 32), jnp.float32),
            jax.ShapeDtypeStruct((_N, 16), jnp.float32),
        ],
    )(node_feat, W_emb, W_msg0, W_vgate0)


def _tc_edgeprep(psrc, pdst, etype2, etype_emb, W_filter):
    def body(ps_ref, pd_ref, et_ref, ee_ref, wf_ref, fc_ref, u32_ref,
             u16_ref):
        rel = pd_ref[...] - ps_ref[...]
        r2 = jnp.sum(rel * rel, axis=1, keepdims=True)
        d = jnp.sqrt(r2 + 1e-12)
        unit = rel / d
        mus = _np.linspace(0.0, _CUTOFF, _EC)
        gamma = 1.0 / (2.0 * (_CUTOFF / _EC) ** 2)
        dm = jnp.concatenate([d - float(m) for m in mus], axis=1)
        rbf = jnp.exp(-gamma * dm * dm)
        cc = 0.5 * (jnp.cos(jnp.pi * jnp.clip(d, 0.0, _CUTOFF) / _CUTOFF)
                    + 1.0)
        wf = wf_ref[...]
        w2 = jnp.dot(ee_ref[...], wf[_EC:, :],
                     preferred_element_type=jnp.float32)
        onehot = (et_ref[...] == jnp.arange(_NT, dtype=jnp.int32)[None, :]
                  ).astype(jnp.float32)
        pre = (jnp.dot(rbf, wf[:_EC, :], preferred_element_type=jnp.float32)
               + jnp.dot(onehot, w2, preferred_element_type=jnp.float32))
        fc_ref[...] = _leaky(pre) * cc
        uc = unit * cc
        e8 = jnp.ones((1, 8), jnp.float32)
        u32_ref[...] = jnp.concatenate(
            [uc[:, 0:1] * e8, uc[:, 1:2] * e8, uc[:, 2:3] * e8,
             jnp.zeros((_BE, 8), jnp.float32)], axis=1)
        u16_ref[...] = jnp.concatenate([cc * e8, cc * e8], axis=1)

    return pl.pallas_call(
        body,
        grid=(_E // _BE,),
        in_specs=[
            pl.BlockSpec((_BE, 4), lambda i: (i, 0)),
            pl.BlockSpec((_BE, 4), lambda i: (i, 0)),
            pl.BlockSpec((_BE, 1), lambda i: (i, 0)),
            pl.BlockSpec((_NT, _EC), lambda i: (0, 0)),
            pl.BlockSpec((2 * _EC, _HS), lambda i: (0, 0)),
        ],
        out_specs=[
            pl.BlockSpec((_BE, _HS), lambda i: (i, 0)),
            pl.BlockSpec((_BE, 32), lambda i: (i, 0)),
            pl.BlockSpec((_BE, 16), lambda i: (i, 0)),
        ],
        out_shape=[
            jax.ShapeDtypeStruct((_E, _HS), jnp.float32),
            jax.ShapeDtypeStruct((_E, 32), jnp.float32),
            jax.ShapeDtypeStruct((_E, 16), jnp.float32),
        ],
    )(psrc, pdst, etype2, etype_emb, W_filter)


def _tc_update_mid(h, hv, aggs, aggv, W_upd_l, Wvmix3_l, W_msg_n, W_vgate_n):
    def body(h_ref, hv_ref, as_ref, av_ref, wu_ref, wm3_ref, wm_ref, wg_ref,
             h2_ref, hv2_ref, gs_ref, gv32_ref, gv16_ref):
        h2 = h_ref[...] + _leaky(jnp.dot(as_ref[...], wu_ref[...],
                                         preferred_element_type=jnp.float32))
        t = hv_ref[...] + av_ref[...][:, 0:24]
        hv2 = jnp.dot(t, wm3_ref[...], preferred_element_type=jnp.float32)
        h2_ref[...] = h2
        hv2_ref[...] = hv2
        gs_ref[...] = jnp.dot(h2, wm_ref[...],
                              preferred_element_type=jnp.float32)
        g = jax.nn.sigmoid(jnp.dot(h2, wg_ref[...],
                                   preferred_element_type=jnp.float32))
        g3 = jnp.concatenate([g, g, g], axis=1)
        gv32_ref[...] = jnp.concatenate([hv2 * g3, g], axis=1)
        gv16_ref[...] = jnp.concatenate([g, g], axis=1)

    return pl.pallas_call(
        body,
        grid=(_N // _BN,),
        in_specs=[
            pl.BlockSpec((_BN, _HS), lambda i: (i, 0)),
            pl.BlockSpec((_BN, 24), lambda i: (i, 0)),
            pl.BlockSpec((_BN, _HS), lambda i: (i, 0)),
            pl.BlockSpec((_BN, _HS), lambda i: (i, 0)),
            pl.BlockSpec((_HS, _HS), lambda i: (0, 0)),
            pl.BlockSpec((24, 24), lambda i: (0, 0)),
            pl.BlockSpec((_HS, _HS), lambda i: (0, 0)),
            pl.BlockSpec((_HS, _HV), lambda i: (0, 0)),
        ],
        out_specs=[
            pl.BlockSpec((_BN, _HS), lambda i: (i, 0)),
            pl.BlockSpec((_BN, 24), lambda i: (i, 0)),
            pl.BlockSpec((_BN, _HS), lambda i: (i, 0)),
            pl.BlockSpec((_BN, 32), lambda i: (i, 0)),
            pl.BlockSpec((_BN, 16), lambda i: (i, 0)),
        ],
        out_shape=[
            jax.ShapeDtypeStruct((_N, _HS), jnp.float32),
            jax.ShapeDtypeStruct((_N, 24), jnp.float32),
            jax.ShapeDtypeStruct((_N, _HS), jnp.float32),
            jax.ShapeDtypeStruct((_N, 32), jnp.float32),
            jax.ShapeDtypeStruct((_N, 16), jnp.float32),
        ],
    )(h, hv, aggs, aggv, W_upd_l, Wvmix3_l, W_msg_n, W_vgate_n)


def _tc_update_last(h, hv, aggs, aggv, W_upd_l, Wvmix3_l):
    def body(h_ref, hv_ref, as_ref, av_ref, wu_ref, wm3_ref, out_ref):
        h2 = h_ref[...] + _leaky(jnp.dot(as_ref[...], wu_ref[...],
                                         preferred_element_type=jnp.float32))
        t = hv_ref[...] + av_ref[...][:, 0:24]
        hv2 = jnp.dot(t, wm3_ref[...], preferred_element_type=jnp.float32)
        cols = [hv2[:, i * _HV + v:i * _HV + v + 1]
                for v in range(_HV) for i in range(3)]
        out_ref[...] = jnp.concatenate([h2] + cols, axis=1)

    return pl.pallas_call(
        body,
        grid=(_N // _BN,),
        in_specs=[
            pl.BlockSpec((_BN, _HS), lambda i: (i, 0)),
            pl.BlockSpec((_BN, 24), lambda i: (i, 0)),
            pl.BlockSpec((_BN, _HS), lambda i: (i, 0)),
            pl.BlockSpec((_BN, _HS), lambda i: (i, 0)),
            pl.BlockSpec((_HS, _HS), lambda i: (0, 0)),
            pl.BlockSpec((24, 24), lambda i: (0, 0)),
        ],
        out_specs=pl.BlockSpec((_BN, 56), lambda i: (i, 0)),
        out_shape=jax.ShapeDtypeStruct((_N, 56), jnp.float32),
    )(h, hv, aggs, aggv, W_upd_l, Wvmix3_l)


def kernel(node_feat, pos, edge_index, edge_type, W_emb, etype_emb, W_filter,
           W_msg, W_upd, W_vgate, W_vmix):
    src = edge_index[0].astype(jnp.int32)
    dst = edge_index[1].astype(jnp.int32)
    etype2 = edge_type.astype(jnp.int32).reshape(_E, 1)
    pos4 = jnp.pad(pos, ((0, 0), (0, 1)))
    zeros32 = jnp.zeros((_N, 32), jnp.float32)
    eye3 = jnp.eye(3, dtype=jnp.float32)
    Wvmix3 = jax.vmap(lambda w: jnp.kron(eye3, w))(W_vmix)

    psrc, pdst = _gather_pos(pos4, src, dst)
    fc, u32, u16 = _tc_edgeprep(psrc, pdst, etype2, etype_emb, W_filter)
    h, gs, gv32, gv16 = _tc_init(node_feat, W_emb, W_msg[0], W_vgate[0])
    hv = jnp.zeros((_N, 24), jnp.float32)

    for l in range(_NL):
        aggs, aggv = _edge_aggregate(gs, gv32, gv16, fc, u32, u16,
                                     src, dst, zeros32)
        if l + 1 < _NL:
            h, hv, gs, gv32, gv16 = _tc_update_mid(
                h, hv, aggs, aggv, W_upd[l], Wvmix3[l],
                W_msg[l + 1], W_vgate[l + 1])
        else:
            out = _tc_update_last(h, hv, aggs, aggv, W_upd[l], Wvmix3[l])
    return out
